# baseline (speedup 1.0000x reference)
import functools

import jax
import jax.numpy as jnp
from jax import lax
from jax.experimental import pallas as pl
from jax.experimental.pallas import tpu as pltpu
from jax.experimental.pallas import tpu_sc as plsc

B = 4
N = 8192
M = 2048
CK = 768
CU = 384
NBLK = 256

NC = 2
NS = 16
NW = NC * NS
L = 16

CHUNKS = NW // B
C_PER_TILE = CK // CHUNKS
U_PER_TILE = CU // 3 // CHUNKS

CB = 4
NH = N // 2
G = C_PER_TILE // CB


def _knn_body(known_ref, unknown_ref, idx_ref, w_ref):
    k = known_ref[0]
    u = unknown_ref[0]
    mm = lax.dot_general(k, u, (((1,), (1,)), ((), ())),
                         precision=lax.Precision.HIGHEST,
                         preferred_element_type=jnp.float32)
    kn2 = jnp.sum(k * k, axis=1, keepdims=True)
    un2 = jnp.sum(u * u, axis=1)[None, :]
    d2 = kn2 - 2.0 * mm + un2
    iot = lax.broadcasted_iota(jnp.int32, d2.shape, 0)
    recips = []
    for t in range(3):
        mv = jnp.min(d2, axis=0, keepdims=True)
        sel = jnp.where(d2 == mv, iot, M)
        mi = jnp.min(sel, axis=0, keepdims=True)
        idx_ref[0, t, :] = mi[0]
        d2 = jnp.where(iot == mi, jnp.float32(jnp.inf), d2)
        dist = jnp.sqrt(jnp.maximum(mv, 0.0))
        recips.append(1.0 / (dist + 1e-8))
    norm = recips[0] + recips[1] + recips[2]
    for t in range(3):
        w_ref[0, t, :] = (recips[t] / norm)[0]


def _three_nn(unknown, known):
    return pl.pallas_call(
        _knn_body,
        grid=(B, N // NBLK),
        in_specs=[
            pl.BlockSpec((1, M, 3), lambda i, j: (i, 0, 0)),
            pl.BlockSpec((1, NBLK, 3), lambda i, j: (i, j, 0)),
        ],
        out_specs=[
            pl.BlockSpec((1, 3, NBLK), lambda i, j: (i, 0, j)),
            pl.BlockSpec((1, 3, NBLK), lambda i, j: (i, 0, j)),
        ],
        out_shape=[
            jax.ShapeDtypeStruct((B, 3, N), jnp.int32),
            jax.ShapeDtypeStruct((B, 3, N), jnp.float32),
        ],
    )(known, unknown)


def _interp_body(kf, idxh, wh, uf, out,
                 idx_v, w_v, tab_v, row_v, cp_v,
                 sem_tab, sem_row, sem_cin, sem_cout):
    cax = lax.axis_index("c")
    sax = lax.axis_index("s")
    wid = sax * NC + cax
    bi = wid // CHUNKS
    ci = lax.rem(wid, CHUNKS)

    cu0 = ci * U_PER_TILE
    NCP = 2 * U_PER_TILE
    NQ = N // 2

    def cp_in(r, slot):
        return pltpu.make_async_copy(
            uf.at[bi, cu0 + r // 2, :, pl.ds(lax.rem(r, 2) * NQ, NQ)],
            cp_v.at[slot], sem_cin.at[slot])

    def cp_out(r, slot):
        return pltpu.make_async_copy(
            cp_v.at[slot],
            out.at[bi, CK // 3 + cu0 + r // 2, :, pl.ds(lax.rem(r, 2) * NQ, NQ)],
            sem_cout.at[slot])

    def cp_step(r, slot):
        @pl.when(r < NCP)
        def _():
            cp_in(r, slot).wait()
            cp_out(r, slot).start()

            @pl.when(r + 1 < NCP)
            def _():
                @pl.when(r >= 1)
                def _():
                    cp_out(r, 1 - slot).wait()

                cp_in(r + 1, 1 - slot).start()

    c0 = ci * C_PER_TILE

    def tab_start(g, slot):
        for c in range(CB):
            ch = c0 + g * CB + c
            pltpu.async_copy(kf.at[bi, ch // 3, pl.ds(lax.rem(ch, 3), 1)],
                             tab_v.at[pl.ds(slot * CB + c, 1)],
                             sem_tab.at[slot])

    def tab_wait(slot):
        for _ in range(CB):
            pltpu.make_async_copy(kf.at[0, 0, pl.ds(0, 1)],
                                  tab_v.at[pl.ds(0, 1)],
                                  sem_tab.at[slot]).wait()

    def row_wait(slot):
        for _ in range(CB):
            pltpu.make_async_copy(row_v.at[pl.ds(0, 1)],
                                  out.at[0, 0, pl.ds(0, 1), pl.ds(0, NH)],
                                  sem_row.at[slot]).wait()

    def group_compute(g, slot):
        tab_wait(slot)
        rows = [slot * CB + c for c in range(CB)]
        rvecs = [jnp.full((L,), r, jnp.int32) for r in rows]

        def j_body(j, carry2):
            off = pl.multiple_of(j * L, L)
            ii = [idx_v[t, pl.ds(off, L)] for t in range(3)]
            ww = [w_v[t, pl.ds(off, L)] for t in range(3)]
            for c in range(CB):
                acc = None
                for t in range(3):
                    g_ = plsc.load_gather(tab_v, [rvecs[c], ii[t]])
                    gw = g_ * ww[t]
                    acc = gw if acc is None else acc + gw
                row_v[rows[c], pl.ds(off, L)] = acc
            return carry2

        lax.fori_loop(0, NH // L, j_body, 0, unroll=2)
        return rows

    def row_start(g, slot, rows, h):
        for c in range(CB):
            ch = c0 + g * CB + c
            pltpu.async_copy(row_v.at[pl.ds(rows[c], 1)],
                             out.at[bi, ch // 3, pl.ds(lax.rem(ch, 3), 1),
                                    pl.ds(h * NH, NH)],
                             sem_row.at[slot])

    cp_in(0, 0).start()

    for h in range(2):
        pltpu.sync_copy(idxh.at[bi, :, pl.ds(h * NH, NH)], idx_v)
        pltpu.sync_copy(wh.at[bi, :, pl.ds(h * NH, NH)], w_v)
        tab_start(0, 0)

        def gp_body(gp, carry):
            for slot in (0, 1):
                g = 2 * gp + slot
                cp_step(h * G + g, slot)

                @pl.when(g < G - 1)
                def _():
                    tab_start(g + 1, 1 - slot)

                @pl.when(g >= 2)
                def _():
                    row_wait(slot)

                rows = group_compute(g, slot)
                row_start(g, slot, rows, h)
            return carry

        lax.fori_loop(0, G // 2, gp_body, 0)
        for slot in (0, 1):
            row_wait(slot)

    for slot in (0, 1):
        cp_out(NCP - 2 + slot, slot).wait()


def _interpolate(kf, idx, w, uf):
    mesh = plsc.VectorSubcoreMesh(core_axis_name="c", subcore_axis_name="s")
    fn = functools.partial(
        pl.kernel,
        out_type=jax.ShapeDtypeStruct((B, CK // 3 + CU // 3, 3, N),
                                      jnp.float32),
        mesh=mesh,
        scratch_types=[
            pltpu.VMEM((3, NH), jnp.int32),
            pltpu.VMEM((3, NH), jnp.float32),
            pltpu.VMEM((2 * CB, M), jnp.float32),
            pltpu.VMEM((2 * CB, NH), jnp.float32),
            pltpu.VMEM((2, 3, N // 2), jnp.float32),
            pltpu.SemaphoreType.DMA((2,)),
            pltpu.SemaphoreType.DMA((2,)),
            pltpu.SemaphoreType.DMA((2,)),
            pltpu.SemaphoreType.DMA((2,)),
        ],
        compiler_params=pltpu.CompilerParams(needs_layout_passes=False,
                                             use_tc_tiling_on_sc=False),
    )(_interp_body)
    return fn(kf, idx, w, uf)


def kernel(unknown, known, unknow_feats, known_feats):
    idx, w = _three_nn(unknown, known)
    return _interpolate(known_feats, idx, w, unknow_feats)


# speedup vs baseline: 1.3219x; 1.3219x over previous
import functools

import jax
import jax.numpy as jnp
from jax import lax
from jax.experimental import pallas as pl
from jax.experimental.pallas import tpu as pltpu
from jax.experimental.pallas import tpu_sc as plsc

B = 4
N = 8192
M = 2048
CK = 768
CU = 384
NBLK = 512

NC = 2
NS = 16
NW = NC * NS
L = 16

CHUNKS = NW // B
C_PER_TILE = CK // CHUNKS
U_PER_TILE = CU // 3 // CHUNKS

CB = 4
NH = N // 2
G = C_PER_TILE // CB


def _knn_body(known_ref, unknown_ref, idx_ref, w_ref):
    k = known_ref[0]
    ut = unknown_ref[0]
    d2 = None
    for d in range(3):
        diff = k[:, d:d + 1] - ut[d:d + 1, :]
        sq = diff * diff
        d2 = sq if d2 is None else d2 + sq
    iot = lax.broadcasted_iota(jnp.int32, d2.shape, 0)
    recips = []
    for t in range(3):
        mv = jnp.min(d2, axis=0, keepdims=True)
        sel = jnp.where(d2 == mv, iot, M)
        mi = jnp.min(sel, axis=0, keepdims=True)
        idx_ref[0, t, :] = mi[0]
        d2 = jnp.where(iot == mi, jnp.float32(jnp.inf), d2)
        dist = jnp.sqrt(jnp.maximum(mv, 0.0))
        recips.append(1.0 / (dist + 1e-8))
    norm = recips[0] + recips[1] + recips[2]
    for t in range(3):
        w_ref[0, t, :] = (recips[t] / norm)[0]


def _three_nn(unknown, known):
    ut = jnp.swapaxes(unknown, 1, 2)
    return pl.pallas_call(
        _knn_body,
        grid=(B, N // NBLK),
        in_specs=[
            pl.BlockSpec((1, M, 3), lambda i, j: (i, 0, 0)),
            pl.BlockSpec((1, 3, NBLK), lambda i, j: (i, 0, j)),
        ],
        out_specs=[
            pl.BlockSpec((1, 3, NBLK), lambda i, j: (i, 0, j)),
            pl.BlockSpec((1, 3, NBLK), lambda i, j: (i, 0, j)),
        ],
        out_shape=[
            jax.ShapeDtypeStruct((B, 3, N), jnp.int32),
            jax.ShapeDtypeStruct((B, 3, N), jnp.float32),
        ],
    )(known, ut)


def _interp_body(kf, idxh, wh, uf, out,
                 idx_v, w_v, tab_v, row_v, cp_v,
                 sem_tab, sem_row, sem_cin, sem_cout):
    cax = lax.axis_index("c")
    sax = lax.axis_index("s")
    wid = sax * NC + cax
    bi = wid // CHUNKS
    ci = lax.rem(wid, CHUNKS)

    cu0 = ci * U_PER_TILE
    NCP = 2 * U_PER_TILE
    NQ = N // 2

    def cp_in(r, slot):
        return pltpu.make_async_copy(
            uf.at[bi, cu0 + r // 2, :, pl.ds(lax.rem(r, 2) * NQ, NQ)],
            cp_v.at[slot], sem_cin.at[slot])

    def cp_out(r, slot):
        return pltpu.make_async_copy(
            cp_v.at[slot],
            out.at[bi, CK // 3 + cu0 + r // 2, :, pl.ds(lax.rem(r, 2) * NQ, NQ)],
            sem_cout.at[slot])

    def cp_step(r, slot):
        @pl.when(r < NCP)
        def _():
            cp_in(r, slot).wait()
            cp_out(r, slot).start()

            @pl.when(r + 1 < NCP)
            def _():
                @pl.when(r >= 1)
                def _():
                    cp_out(r, 1 - slot).wait()

                cp_in(r + 1, 1 - slot).start()

    c0 = ci * C_PER_TILE

    def tab_start(g, slot):
        for c in range(CB):
            ch = c0 + g * CB + c
            pltpu.async_copy(kf.at[bi, ch // 3, pl.ds(lax.rem(ch, 3), 1)],
                             tab_v.at[pl.ds(slot * CB + c, 1)],
                             sem_tab.at[slot])

    def tab_wait(slot):
        for _ in range(CB):
            pltpu.make_async_copy(kf.at[0, 0, pl.ds(0, 1)],
                                  tab_v.at[pl.ds(0, 1)],
                                  sem_tab.at[slot]).wait()

    def row_wait(slot):
        for _ in range(CB):
            pltpu.make_async_copy(row_v.at[pl.ds(0, 1)],
                                  out.at[0, 0, pl.ds(0, 1), pl.ds(0, NH)],
                                  sem_row.at[slot]).wait()

    def group_compute(g, slot):
        tab_wait(slot)
        rows = [slot * CB + c for c in range(CB)]
        rvecs = [jnp.full((L,), r, jnp.int32) for r in rows]

        def j_body(j, carry2):
            off = pl.multiple_of(j * L, L)
            ii = [idx_v[t, pl.ds(off, L)] for t in range(3)]
            ww = [w_v[t, pl.ds(off, L)] for t in range(3)]
            for c in range(CB):
                acc = None
                for t in range(3):
                    g_ = plsc.load_gather(tab_v, [rvecs[c], ii[t]])
                    gw = g_ * ww[t]
                    acc = gw if acc is None else acc + gw
                row_v[rows[c], pl.ds(off, L)] = acc
            return carry2

        lax.fori_loop(0, NH // L, j_body, 0, unroll=2)
        return rows

    def row_start(g, slot, rows, h):
        for c in range(CB):
            ch = c0 + g * CB + c
            pltpu.async_copy(row_v.at[pl.ds(rows[c], 1)],
                             out.at[bi, ch // 3, pl.ds(lax.rem(ch, 3), 1),
                                    pl.ds(h * NH, NH)],
                             sem_row.at[slot])

    cp_in(0, 0).start()

    for h in range(2):
        pltpu.sync_copy(idxh.at[bi, :, pl.ds(h * NH, NH)], idx_v)
        pltpu.sync_copy(wh.at[bi, :, pl.ds(h * NH, NH)], w_v)
        tab_start(0, 0)

        def gp_body(gp, carry):
            for slot in (0, 1):
                g = 2 * gp + slot
                cp_step(h * G + g, slot)

                @pl.when(g < G - 1)
                def _():
                    tab_start(g + 1, 1 - slot)

                @pl.when(g >= 2)
                def _():
                    row_wait(slot)

                rows = group_compute(g, slot)
                row_start(g, slot, rows, h)
            return carry

        lax.fori_loop(0, G // 2, gp_body, 0)
        for slot in (0, 1):
            row_wait(slot)

    for slot in (0, 1):
        cp_out(NCP - 2 + slot, slot).wait()


def _interpolate(kf, idx, w, uf):
    mesh = plsc.VectorSubcoreMesh(core_axis_name="c", subcore_axis_name="s")
    fn = functools.partial(
        pl.kernel,
        out_type=jax.ShapeDtypeStruct((B, CK // 3 + CU // 3, 3, N),
                                      jnp.float32),
        mesh=mesh,
        scratch_types=[
            pltpu.VMEM((3, NH), jnp.int32),
            pltpu.VMEM((3, NH), jnp.float32),
            pltpu.VMEM((2 * CB, M), jnp.float32),
            pltpu.VMEM((2 * CB, NH), jnp.float32),
            pltpu.VMEM((2, 3, N // 2), jnp.float32),
            pltpu.SemaphoreType.DMA((2,)),
            pltpu.SemaphoreType.DMA((2,)),
            pltpu.SemaphoreType.DMA((2,)),
            pltpu.SemaphoreType.DMA((2,)),
        ],
        compiler_params=pltpu.CompilerParams(needs_layout_passes=False),
    )(_interp_body)
    return fn(kf, idx, w, uf)


def kernel(unknown, known, unknow_feats, known_feats):
    idx, w = _three_nn(unknown, known)
    return _interpolate(known_feats, idx, w, unknow_feats)


# speedup vs baseline: 1.3628x; 1.0310x over previous
import functools

import jax
import jax.numpy as jnp
from jax import lax
from jax.experimental import pallas as pl
from jax.experimental.pallas import tpu as pltpu
from jax.experimental.pallas import tpu_sc as plsc

B = 4
N = 8192
M = 2048
CK = 768
CU = 384
NBLK = 512

NC = 2
NS = 16
NW = NC * NS
L = 16

CHUNKS = NW // B
C_PER_TILE = CK // CHUNKS
U_PER_TILE = CU // 3 // CHUNKS

CB = 4
NH = N // 2
G = C_PER_TILE // CB


def _knn_body(known_ref, unknown_ref, idx_ref, w_ref):
    k = known_ref[0]
    ut = unknown_ref[0]
    d2 = None
    for d in range(3):
        diff = k[:, d:d + 1] - ut[d:d + 1, :]
        sq = diff * diff
        d2 = sq if d2 is None else d2 + sq
    iot = lax.broadcasted_iota(jnp.int32, d2.shape, 0)
    recips = []
    for t in range(3):
        mv = jnp.min(d2, axis=0, keepdims=True)
        sel = jnp.where(d2 == mv, iot, M)
        mi = jnp.min(sel, axis=0, keepdims=True)
        idx_ref[0, t, :] = mi[0]
        d2 = jnp.where(iot == mi, jnp.float32(jnp.inf), d2)
        dist = jnp.sqrt(jnp.maximum(mv, 0.0))
        recips.append(1.0 / (dist + 1e-8))
    norm = recips[0] + recips[1] + recips[2]
    for t in range(3):
        w_ref[0, t, :] = (recips[t] / norm)[0]


def _three_nn(unknown, known):
    ut = jnp.swapaxes(unknown, 1, 2)
    return pl.pallas_call(
        _knn_body,
        grid=(B, N // NBLK),
        in_specs=[
            pl.BlockSpec((1, M, 3), lambda i, j: (i, 0, 0)),
            pl.BlockSpec((1, 3, NBLK), lambda i, j: (i, 0, j)),
        ],
        out_specs=[
            pl.BlockSpec((1, 3, NBLK), lambda i, j: (i, 0, j)),
            pl.BlockSpec((1, 3, NBLK), lambda i, j: (i, 0, j)),
        ],
        out_shape=[
            jax.ShapeDtypeStruct((B, 3, N), jnp.int32),
            jax.ShapeDtypeStruct((B, 3, N), jnp.float32),
        ],
    )(known, ut)


def _interp_body(kf, idxh, wh, uf, out,
                 idx_v, w_v, tab_v, row_v, cp_v,
                 sem_tab, sem_row, sem_cin, sem_cout):
    cax = lax.axis_index("c")
    sax = lax.axis_index("s")
    wid = sax * NC + cax
    bi = wid // CHUNKS
    ci = lax.rem(wid, CHUNKS)

    cu0 = ci * U_PER_TILE
    NCP = 2 * U_PER_TILE
    NQ = N // 2

    def cp_in(r, slot):
        return pltpu.make_async_copy(
            uf.at[bi, cu0 + r // 2, :, pl.ds(lax.rem(r, 2) * NQ, NQ)],
            cp_v.at[slot], sem_cin.at[slot])

    def cp_out(r, slot):
        return pltpu.make_async_copy(
            cp_v.at[slot],
            out.at[bi, CK // 3 + cu0 + r // 2, :, pl.ds(lax.rem(r, 2) * NQ, NQ)],
            sem_cout.at[slot])

    def cp_step(r, slot):
        @pl.when(r < NCP)
        def _():
            cp_in(r, slot).wait()
            cp_out(r, slot).start()

            @pl.when(r + 1 < NCP)
            def _():
                @pl.when(r >= 1)
                def _():
                    cp_out(r, 1 - slot).wait()

                cp_in(r + 1, 1 - slot).start()

    c0 = ci * C_PER_TILE

    def tab_start(g, slot):
        for c in range(CB):
            ch = c0 + g * CB + c
            pltpu.async_copy(kf.at[bi, ch // 3, pl.ds(lax.rem(ch, 3), 1)],
                             tab_v.at[:, pl.ds((slot * CB + c) * M, M)],
                             sem_tab.at[slot])

    def tab_wait(slot):
        for _ in range(CB):
            pltpu.make_async_copy(kf.at[0, 0, pl.ds(0, 1)],
                                  tab_v.at[:, pl.ds(0, M)],
                                  sem_tab.at[slot]).wait()

    def row_wait(slot):
        for _ in range(CB):
            pltpu.make_async_copy(row_v.at[pl.ds(0, 1)],
                                  out.at[0, 0, pl.ds(0, 1), pl.ds(0, NH)],
                                  sem_row.at[slot]).wait()

    def group_compute(g, slot):
        tab_wait(slot)
        rows = [slot * CB + c for c in range(CB)]
        zvec = jnp.zeros((L,), jnp.int32)

        def j_body(j, carry2):
            off = pl.multiple_of(j * L, L)
            ii = [idx_v[t, pl.ds(off, L)] for t in range(3)]
            ww = [w_v[t, pl.ds(off, L)] for t in range(3)]
            for c in range(CB):
                acc = None
                for t in range(3):
                    g_ = plsc.load_gather(tab_v, [zvec, ii[t] + rows[c] * M])
                    gw = g_ * ww[t]
                    acc = gw if acc is None else acc + gw
                row_v[rows[c], pl.ds(off, L)] = acc
            return carry2

        lax.fori_loop(0, NH // L, j_body, 0, unroll=2)
        return rows

    def row_start(g, slot, rows, h):
        for c in range(CB):
            ch = c0 + g * CB + c
            pltpu.async_copy(row_v.at[pl.ds(rows[c], 1)],
                             out.at[bi, ch // 3, pl.ds(lax.rem(ch, 3), 1),
                                    pl.ds(h * NH, NH)],
                             sem_row.at[slot])

    cp_in(0, 0).start()

    for h in range(2):
        pltpu.sync_copy(idxh.at[bi, :, pl.ds(h * NH, NH)], idx_v)
        pltpu.sync_copy(wh.at[bi, :, pl.ds(h * NH, NH)], w_v)
        tab_start(0, 0)

        def gp_body(gp, carry):
            for slot in (0, 1):
                g = 2 * gp + slot
                cp_step(h * G + g, slot)

                @pl.when(g < G - 1)
                def _():
                    tab_start(g + 1, 1 - slot)

                @pl.when(g >= 2)
                def _():
                    row_wait(slot)

                rows = group_compute(g, slot)
                row_start(g, slot, rows, h)
            return carry

        lax.fori_loop(0, G // 2, gp_body, 0)
        for slot in (0, 1):
            row_wait(slot)

    for slot in (0, 1):
        cp_out(NCP - 2 + slot, slot).wait()


def _interpolate(kf, idx, w, uf):
    mesh = plsc.VectorSubcoreMesh(core_axis_name="c", subcore_axis_name="s")
    fn = functools.partial(
        pl.kernel,
        out_type=jax.ShapeDtypeStruct((B, CK // 3 + CU // 3, 3, N),
                                      jnp.float32),
        mesh=mesh,
        scratch_types=[
            pltpu.VMEM((3, NH), jnp.int32),
            pltpu.VMEM((3, NH), jnp.float32),
            pltpu.VMEM((1, 2 * CB * M), jnp.float32),
            pltpu.VMEM((2 * CB, NH), jnp.float32),
            pltpu.VMEM((2, 3, N // 2), jnp.float32),
            pltpu.SemaphoreType.DMA((2,)),
            pltpu.SemaphoreType.DMA((2,)),
            pltpu.SemaphoreType.DMA((2,)),
            pltpu.SemaphoreType.DMA((2,)),
        ],
        compiler_params=pltpu.CompilerParams(needs_layout_passes=False),
    )(_interp_body)
    return fn(kf, idx, w, uf)


def kernel(unknown, known, unknow_feats, known_feats):
    idx, w = _three_nn(unknown, known)
    return _interpolate(known_feats, idx, w, unknow_feats)
